# baseline (speedup 1.0000x reference)
import functools

import jax
import jax.numpy as jnp
from jax import lax
from jax.experimental import pallas as pl
from jax.experimental.pallas import tpu as pltpu
from jax.experimental.pallas import tpu_sc as plsc

_N = 16384
_C = 10
_BJ = 2048
_LS = 0.1
_LAM = 0.5
_B = 2048
_LO = -8.0
_SCALE = _B / 8.0
_NV = _N // 16


def _prep_body(x_ref, t_ref, s_ref, stat_ref):
    j = pl.program_id(0)
    x = x_ref[...]
    m = jnp.max(x, axis=0, keepdims=True)
    e = jnp.exp(x - m)
    tot = jnp.sum(e, axis=0, keepdims=True)
    s = (x - m) - jnp.log(tot)
    s_ref[...] = s

    t = t_ref[pl.ds(j * _BJ, _BJ)]
    cls_col = lax.broadcasted_iota(jnp.int32, (_C, _BJ), 0)
    ht = (cls_col == t[None, :]).astype(jnp.float32)
    o = jnp.sum(ht * s, axis=0)
    colsum = jnp.sum(s, axis=0)
    ce_part = jnp.sum(-((1.0 - _LS) * o + (_LS / _C) * colsum))

    row = lax.broadcasted_iota(jnp.int32, (8, 128), 0)
    upd = jnp.where(row == 0, ce_part, 0.0)

    @pl.when(j == 0)
    def _():
        stat_ref[...] = jnp.zeros((8, 128), jnp.float32)

    stat_ref[...] += upd


def _bucketize(k):
    b = ((k - _LO) * _SCALE).astype(jnp.int32)
    return jnp.clip(b, 0, _B - 1)


def _sc_class_work(c, role, row, s_hbm, t_hbm, out_hbm, col, tloc, hist,
                   bsum, cumx, pbsum, pcumx, pcurs, psort, pk, pbk, stage,
                   vtmp):
    lanes = lax.iota(jnp.int32, 16)
    zeros16i = jnp.zeros(16, jnp.int32)
    ones16i = jnp.ones(16, jnp.int32)

    pltpu.sync_copy(s_hbm.at[c], col)
    pltpu.sync_copy(t_hbm, tloc)

    def z_body(i, carry):
        for u in range(4):
            hist[pl.ds((i * 4 + u) * 16, 16)] = zeros16i
        return carry

    lax.fori_loop(0, _B // 4, z_body, 0)

    def h_body(v4, carry):
        for u in range(4):
            k = col[pl.ds((v4 * 4 + u) * 16, 16)]
            b = _bucketize(k)
            plsc.addupdate_scatter(hist, [lanes * _B + b], ones16i)
        return carry

    lax.fori_loop(0, _NV // 4, h_body, 0)

    def r_body(i, carry):
        acc = hist[pl.ds(i * 16, 16)]
        for l in range(1, 16):
            acc = acc + hist[pl.ds(l * _B + i * 16, 16)]
        bsum[pl.ds(i * 16, 16)] = acc
        return carry

    lax.fori_loop(0, _B // 16, r_body, 0)

    def c_body(i, carry):
        v = bsum[pl.ds(i * 16, 16)]
        inc = plsc.cumsum(v)
        cumx[pl.ds(i * 16, 16)] = (inc - v) + carry
        return carry + jnp.max(inc)

    lax.fori_loop(0, _B // 16, c_body, 0)

    lax.fori_loop(0, _B // 4, z_body, 0)

    def p_body(v, carry):
        npos, term1 = carry
        k = col[pl.ds(v * 16, 16)]
        t = tloc[pl.ds(v * 16, 16)]
        b = _bucketize(k)
        pos = t == c
        base = plsc.load_gather(cumx, [b])
        term1 = term1 + jnp.where(pos & (role == 0), base, 0)
        plsc.store_compressed(pk.at[pl.ds(npos, 16)], k, mask=pos)
        plsc.store_compressed(pbk.at[pl.ds(npos, 16)], b, mask=pos)
        plsc.addupdate_scatter(hist, [lanes * _B + b], ones16i, mask=pos)
        cnt = jnp.max(plsc.all_reduce_population_count(pos))
        return npos + cnt, term1

    npos, term1 = lax.fori_loop(0, _NV, p_body, (jnp.int32(0), zeros16i))

    lax.fori_loop(0, _B // 16, r_body, 0)

    def pc_body(i, carry):
        v = bsum[pl.ds(i * 16, 16)]
        inc = plsc.cumsum(v)
        exc = (inc - v) + carry
        pbsum[pl.ds(i * 16, 16)] = v
        pcumx[pl.ds(i * 16, 16)] = exc
        pcurs[pl.ds(i * 16, 16)] = exc
        return carry + jnp.max(inc)

    lax.fori_loop(0, _B // 16, pc_body, 0)

    nfull = npos // 16

    def m_body(v, carry):
        b = pbk[pl.ds(v * 16, 16)]
        k = pk[pl.ds(v * 16, 16)]
        b_s, k_s = plsc.sort_key_val(b, k)
        vtmp[...] = b_s
        prev = plsc.load_gather(vtmp, [jnp.maximum(lanes - 1, 0)])
        nxt = plsc.load_gather(vtmp, [jnp.minimum(lanes + 1, 15)])
        start = (lanes == 0) | (b_s != prev)
        end = (lanes == 15) | (b_s != nxt)
        ordv = lanes - plsc.cummax(jnp.where(start, lanes, 0))
        cur = plsc.load_gather(pcurs, [b_s])
        addr = cur + ordv
        plsc.store_scatter(psort, [addr], k_s)
        plsc.store_scatter(pcurs, [b_s], addr + 1, mask=end)
        return carry

    lax.fori_loop(0, nfull, m_body, 0)

    tb = pbk[pl.ds(nfull * 16, 16)]
    tk = pk[pl.ds(nfull * 16, 16)]

    def tail_body(l, carry):
        lm = (lanes == l) & (nfull * 16 + l < npos)
        cur = plsc.load_gather(pcurs, [jnp.where(lm, tb, 0)])
        plsc.store_scatter(psort, [cur], tk, mask=lm)
        plsc.store_scatter(pcurs, [tb], cur + 1, mask=lm)
        return carry

    lax.fori_loop(0, 16, tail_body, 0)

    ntrip = (_NV - role + 2) // 3

    def q_body(v, t2):
        vv = role + 3 * v
        k = col[pl.ds(vv * 16, 16)]
        b = _bucketize(k)
        base = plsc.load_gather(pcumx, [b])
        n = plsc.load_gather(pbsum, [b])
        nmax = jnp.max(n)

        def inner(m4, acc):
            for u in range(4):
                m = m4 * 4 + u
                mm = m < n
                idx = jnp.minimum(base + m, _N - 1)
                g = plsc.load_gather(psort, [idx], mask=mm)
                acc = acc + jnp.where(mm & (g > k), 1, 0)
            return acc

        return lax.fori_loop(0, (nmax + 3) // 4, inner, t2)

    t2 = lax.fori_loop(0, ntrip, q_body, zeros16i)

    r_vec = term1.astype(jnp.float32) + t2.astype(jnp.float32)
    stage[pl.ds(0, 16)] = r_vec
    pw = jnp.where((lanes == 0) & (role == 0), npos.astype(jnp.float32), 0.0)
    stage[pl.ds(16, 16)] = pw
    pltpu.sync_copy(stage, out_hbm.at[row])


def _sc_count_body(s_hbm, t_hbm, out_hbm, col, tloc, hist, bsum, cumx,
                   pbsum, pcumx, pcurs, psort, pk, pbk, stage, vtmp):
    wid = lax.axis_index("s") * 2 + lax.axis_index("c")

    @pl.when(wid < 3 * _C)
    def _():
        _sc_class_work(wid % _C, wid // _C, wid, s_hbm, t_hbm, out_hbm, col,
                       tloc, hist, bsum, cumx, pbsum, pcumx, pcurs, psort,
                       pk, pbk, stage, vtmp)


@functools.cache
def _sc_count():
  return functools.partial(
    pl.kernel,
    out_type=jax.ShapeDtypeStruct((3 * _C, 32), jnp.float32),
    mesh=plsc.VectorSubcoreMesh(core_axis_name="c", subcore_axis_name="s"),
    compiler_params=pltpu.CompilerParams(needs_layout_passes=False),
    scratch_types=[
        pltpu.VMEM((_N,), jnp.float32),
        pltpu.VMEM((_N,), jnp.int32),
        pltpu.VMEM((_B * 16,), jnp.int32),
        pltpu.VMEM((_B,), jnp.int32),
        pltpu.VMEM((_B,), jnp.int32),
        pltpu.VMEM((_B,), jnp.int32),
        pltpu.VMEM((_B,), jnp.int32),
        pltpu.VMEM((_B,), jnp.int32),
        pltpu.VMEM((_N,), jnp.float32),
        pltpu.VMEM((_N + 16,), jnp.float32),
        pltpu.VMEM((_N + 16,), jnp.int32),
        pltpu.VMEM((32,), jnp.float32),
        pltpu.VMEM((16,), jnp.int32),
    ],
  )(_sc_count_body)


def _final_body(stat_ref, rp_ref, out_ref):
    ce_sum = stat_ref[0, 0]
    rp = rp_ref[...]
    lane = lax.broadcasted_iota(jnp.int32, (3 * _C, 32), 1)
    rw = jnp.sum(jnp.where(lane < 16, rp, 0.0), axis=1)
    pw = jnp.sum(jnp.where(lane >= 16, rp, 0.0), axis=1)
    r = rw[0:_C] + rw[_C:2 * _C] + rw[2 * _C:3 * _C]
    p = pw[0:_C] + pw[_C:2 * _C] + pw[2 * _C:3 * _C]
    n = _N - p
    u = r - p * (p - 1.0) * 0.5
    denom = jnp.maximum(p, 1.0) * jnp.maximum(n, 1.0)
    pauc = jnp.sum(u / denom) / _C
    ce = ce_sum / _N
    out_ref[0, 0] = (1.0 - _LAM) * ce + _LAM * (1.0 - pauc * pauc)


@jax.jit
def kernel(predictions, targets):
    pred_t = predictions.T

    s, stat = pl.pallas_call(
        _prep_body,
        grid=(_N // _BJ,),
        in_specs=[
            pl.BlockSpec((_C, _BJ), lambda j: (0, j)),
            pl.BlockSpec((_N,), lambda j: (0,)),
        ],
        out_specs=[
            pl.BlockSpec((_C, _BJ), lambda j: (0, j)),
            pl.BlockSpec((8, 128), lambda j: (0, 0)),
        ],
        out_shape=[
            jax.ShapeDtypeStruct((_C, _N), jnp.float32),
            jax.ShapeDtypeStruct((8, 128), jnp.float32),
        ],
    )(pred_t, targets)

    rp = _sc_count()(s, targets)

    out = pl.pallas_call(
        _final_body,
        in_specs=[
            pl.BlockSpec((8, 128), lambda: (0, 0)),
            pl.BlockSpec((3 * _C, 32), lambda: (0, 0)),
        ],
        out_specs=pl.BlockSpec((1, 1), lambda: (0, 0), memory_space=pltpu.SMEM),
        out_shape=jax.ShapeDtypeStruct((1, 1), jnp.float32),
    )(stat, rp)

    return out[0, 0]


# speedup vs baseline: 1.0262x; 1.0262x over previous
import functools

import jax
import jax.numpy as jnp
from jax import lax
from jax.experimental import pallas as pl
from jax.experimental.pallas import tpu as pltpu
from jax.experimental.pallas import tpu_sc as plsc

_N = 16384
_C = 10
_BJ = 2048
_LS = 0.1
_LAM = 0.5
_B = 2048
_LO = -8.0
_SCALE = _B / 8.0
_NV = _N // 16


def _prep_body(x_ref, t_ref, s_ref, stat_ref):
    j = pl.program_id(0)
    x = x_ref[...]
    m = jnp.max(x, axis=0, keepdims=True)
    e = jnp.exp(x - m)
    tot = jnp.sum(e, axis=0, keepdims=True)
    s = (x - m) - jnp.log(tot)
    s_ref[...] = s

    t = t_ref[pl.ds(j * _BJ, _BJ)]
    cls_col = lax.broadcasted_iota(jnp.int32, (_C, _BJ), 0)
    ht = (cls_col == t[None, :]).astype(jnp.float32)
    o = jnp.sum(ht * s, axis=0)
    colsum = jnp.sum(s, axis=0)
    ce_part = jnp.sum(-((1.0 - _LS) * o + (_LS / _C) * colsum))

    row = lax.broadcasted_iota(jnp.int32, (8, 128), 0)
    upd = jnp.where(row == 0, ce_part, 0.0)

    @pl.when(j == 0)
    def _():
        stat_ref[...] = jnp.zeros((8, 128), jnp.float32)

    stat_ref[...] += upd


def _bucketize(k):
    b = ((k - _LO) * _SCALE).astype(jnp.int32)
    return jnp.clip(b, 0, _B - 1)


def _sc_class_work(c, role, row, s_hbm, t_hbm, out_hbm, col, tloc, hist,
                   bsum, cumx, pbsum, pcumx, pcurs, psort, pk, pbk, stage,
                   vtmp, sem1, sem2):
    lanes = lax.iota(jnp.int32, 16)
    zeros16i = jnp.zeros(16, jnp.int32)
    ones16i = jnp.ones(16, jnp.int32)

    cp1 = pltpu.async_copy(s_hbm.at[c], col, sem1)
    cp2 = pltpu.async_copy(t_hbm, tloc, sem2)

    def z_body(i, carry):
        for u in range(4):
            hist[pl.ds((i * 4 + u) * 16, 16)] = zeros16i
        return carry

    lax.fori_loop(0, _B // 4, z_body, 0)
    cp1.wait()
    cp2.wait()

    def h_body(v4, carry):
        for u in range(4):
            k = col[pl.ds((v4 * 4 + u) * 16, 16)]
            b = _bucketize(k)
            plsc.addupdate_scatter(hist, [lanes * _B + b], ones16i)
        return carry

    lax.fori_loop(0, _NV // 4, h_body, 0)

    def r_body(i, carry):
        acc = hist[pl.ds(i * 16, 16)]
        for l in range(1, 16):
            acc = acc + hist[pl.ds(l * _B + i * 16, 16)]
        bsum[pl.ds(i * 16, 16)] = acc
        return carry

    lax.fori_loop(0, _B // 16, r_body, 0)

    def c_body(i, carry):
        v = bsum[pl.ds(i * 16, 16)]
        inc = plsc.cumsum(v)
        cumx[pl.ds(i * 16, 16)] = (inc - v) + carry
        return carry + jnp.max(inc)

    lax.fori_loop(0, _B // 16, c_body, 0)

    lax.fori_loop(0, _B // 4, z_body, 0)

    def p_body(v, carry):
        npos, term1 = carry
        k = col[pl.ds(v * 16, 16)]
        t = tloc[pl.ds(v * 16, 16)]
        b = _bucketize(k)
        pos = t == c
        base = plsc.load_gather(cumx, [b])
        term1 = term1 + jnp.where(pos & (role == 0), base, 0)
        plsc.store_compressed(pk.at[pl.ds(npos, 16)], k, mask=pos)
        plsc.store_compressed(pbk.at[pl.ds(npos, 16)], b, mask=pos)
        plsc.addupdate_scatter(hist, [lanes * _B + b], ones16i, mask=pos)
        cnt = jnp.max(plsc.all_reduce_population_count(pos))
        return npos + cnt, term1

    npos, term1 = lax.fori_loop(0, _NV, p_body, (jnp.int32(0), zeros16i))

    lax.fori_loop(0, _B // 16, r_body, 0)

    def pc_body(i, carry):
        v = bsum[pl.ds(i * 16, 16)]
        inc = plsc.cumsum(v)
        exc = (inc - v) + carry
        pbsum[pl.ds(i * 16, 16)] = v
        pcumx[pl.ds(i * 16, 16)] = exc
        pcurs[pl.ds(i * 16, 16)] = exc
        return carry + jnp.max(inc)

    lax.fori_loop(0, _B // 16, pc_body, 0)

    nfull = npos // 16

    def m_body(v, carry):
        b = pbk[pl.ds(v * 16, 16)]
        k = pk[pl.ds(v * 16, 16)]
        b_s, k_s = plsc.sort_key_val(b, k)
        vtmp[...] = b_s
        prev = plsc.load_gather(vtmp, [jnp.maximum(lanes - 1, 0)])
        nxt = plsc.load_gather(vtmp, [jnp.minimum(lanes + 1, 15)])
        start = (lanes == 0) | (b_s != prev)
        end = (lanes == 15) | (b_s != nxt)
        ordv = lanes - plsc.cummax(jnp.where(start, lanes, 0))
        cur = plsc.load_gather(pcurs, [b_s])
        addr = cur + ordv
        plsc.store_scatter(psort, [addr], k_s)
        plsc.store_scatter(pcurs, [b_s], addr + 1, mask=end)
        return carry

    lax.fori_loop(0, nfull, m_body, 0)

    tb = pbk[pl.ds(nfull * 16, 16)]
    tk = pk[pl.ds(nfull * 16, 16)]

    def tail_body(l, carry):
        lm = (lanes == l) & (nfull * 16 + l < npos)
        cur = plsc.load_gather(pcurs, [jnp.where(lm, tb, 0)])
        plsc.store_scatter(psort, [cur], tk, mask=lm)
        plsc.store_scatter(pcurs, [tb], cur + 1, mask=lm)
        return carry

    lax.fori_loop(0, 16, tail_body, 0)

    ntrip = (_NV - role + 2) // 3

    def q_body(v, t2):
        vv = role + 3 * v
        k = col[pl.ds(vv * 16, 16)]
        b = _bucketize(k)
        base = plsc.load_gather(pcumx, [b])
        n = plsc.load_gather(pbsum, [b])
        nmax = jnp.max(n)

        def inner(m4, acc):
            for u in range(4):
                m = m4 * 4 + u
                mm = m < n
                idx = jnp.minimum(base + m, _N - 1)
                g = plsc.load_gather(psort, [idx], mask=mm)
                acc = acc + jnp.where(mm & (g > k), 1, 0)
            return acc

        return lax.fori_loop(0, (nmax + 3) // 4, inner, t2)

    t2 = lax.fori_loop(0, ntrip, q_body, zeros16i)

    r_vec = term1.astype(jnp.float32) + t2.astype(jnp.float32)
    stage[pl.ds(0, 16)] = r_vec
    pw = jnp.where((lanes == 0) & (role == 0), npos.astype(jnp.float32), 0.0)
    stage[pl.ds(16, 16)] = pw
    pltpu.sync_copy(stage, out_hbm.at[row])


def _sc_count_body(s_hbm, t_hbm, out_hbm, col, tloc, hist, bsum, cumx,
                   pbsum, pcumx, pcurs, psort, pk, pbk, stage, vtmp, sem1,
                   sem2):
    wid = lax.axis_index("s") * 2 + lax.axis_index("c")

    @pl.when(wid < 3 * _C)
    def _():
        _sc_class_work(wid % _C, wid // _C, wid, s_hbm, t_hbm, out_hbm, col,
                       tloc, hist, bsum, cumx, pbsum, pcumx, pcurs, psort,
                       pk, pbk, stage, vtmp, sem1, sem2)


@functools.cache
def _sc_count():
  return functools.partial(
    pl.kernel,
    out_type=jax.ShapeDtypeStruct((3 * _C, 32), jnp.float32),
    mesh=plsc.VectorSubcoreMesh(core_axis_name="c", subcore_axis_name="s"),
    compiler_params=pltpu.CompilerParams(needs_layout_passes=False),
    scratch_types=[
        pltpu.VMEM((_N,), jnp.float32),
        pltpu.VMEM((_N,), jnp.int32),
        pltpu.VMEM((_B * 16,), jnp.int32),
        pltpu.VMEM((_B,), jnp.int32),
        pltpu.VMEM((_B,), jnp.int32),
        pltpu.VMEM((_B,), jnp.int32),
        pltpu.VMEM((_B,), jnp.int32),
        pltpu.VMEM((_B,), jnp.int32),
        pltpu.VMEM((_N,), jnp.float32),
        pltpu.VMEM((_N + 16,), jnp.float32),
        pltpu.VMEM((_N + 16,), jnp.int32),
        pltpu.VMEM((32,), jnp.float32),
        pltpu.VMEM((16,), jnp.int32),
        pltpu.SemaphoreType.DMA,
        pltpu.SemaphoreType.DMA,
    ],
  )(_sc_count_body)


def _final_body(stat_ref, rp_ref, out_ref):
    ce_sum = stat_ref[0, 0]
    rp = rp_ref[...]
    lane = lax.broadcasted_iota(jnp.int32, (3 * _C, 32), 1)
    rw = jnp.sum(jnp.where(lane < 16, rp, 0.0), axis=1)
    pw = jnp.sum(jnp.where(lane >= 16, rp, 0.0), axis=1)
    r = rw[0:_C] + rw[_C:2 * _C] + rw[2 * _C:3 * _C]
    p = pw[0:_C] + pw[_C:2 * _C] + pw[2 * _C:3 * _C]
    n = _N - p
    u = r - p * (p - 1.0) * 0.5
    denom = jnp.maximum(p, 1.0) * jnp.maximum(n, 1.0)
    pauc = jnp.sum(u / denom) / _C
    ce = ce_sum / _N
    out_ref[0, 0] = (1.0 - _LAM) * ce + _LAM * (1.0 - pauc * pauc)


@jax.jit
def kernel(predictions, targets):
    pred_t = predictions.T

    s, stat = pl.pallas_call(
        _prep_body,
        grid=(_N // _BJ,),
        in_specs=[
            pl.BlockSpec((_C, _BJ), lambda j: (0, j)),
            pl.BlockSpec((_N,), lambda j: (0,)),
        ],
        out_specs=[
            pl.BlockSpec((_C, _BJ), lambda j: (0, j)),
            pl.BlockSpec((8, 128), lambda j: (0, 0)),
        ],
        out_shape=[
            jax.ShapeDtypeStruct((_C, _N), jnp.float32),
            jax.ShapeDtypeStruct((8, 128), jnp.float32),
        ],
    )(pred_t, targets)

    rp = _sc_count()(s, targets)

    out = pl.pallas_call(
        _final_body,
        in_specs=[
            pl.BlockSpec((8, 128), lambda: (0, 0)),
            pl.BlockSpec((3 * _C, 32), lambda: (0, 0)),
        ],
        out_specs=pl.BlockSpec((1, 1), lambda: (0, 0), memory_space=pltpu.SMEM),
        out_shape=jax.ShapeDtypeStruct((1, 1), jnp.float32),
    )(stat, rp)

    return out[0, 0]
